# Initial kernel scaffold; baseline (speedup 1.0000x reference)
import jax
import jax.numpy as jnp
from jax.experimental import pallas as pl


def kernel(x, codebooks):
    raise NotImplementedError("write your pallas kernel here")


# speedup vs baseline: 1.2741x; 1.2741x over previous
import jax
import jax.numpy as jnp
from jax.experimental import pallas as pl
from jax.experimental.pallas import tpu as pltpu
from jax.experimental.pallas import tpu_sc as plsc

NQ = 4
K = 8192
D = 256
B, S = 32, 576
M = B * S
TM = 256
GRID_M = M // TM
GW = 128

_TC_PARAMS = pltpu.CompilerParams(dimension_semantics=("parallel",))


def _b2_body(cb_ref, b2_ref):
    for q in range(NQ):
        cb = cb_ref[q]
        b2_ref[q] = jnp.sum(cb * cb, axis=-1)[None, :]


def _codebook_sqnorms(codebooks):
    return pl.pallas_call(
        _b2_body,
        out_shape=jax.ShapeDtypeStruct((NQ, 1, K), jnp.float32),
    )(codebooks)


def _first_step_body(x_ref, cb_ref, b2_ref, idx_ref):
    res = x_ref[...]
    a2 = jnp.sum(res * res, axis=-1, keepdims=True)
    scores = jax.lax.dot_general(
        res, cb_ref[...], (((1,), (1,)), ((), ())),
        preferred_element_type=jnp.float32)
    d2 = (a2 - 2.0 * scores) + b2_ref[...]
    d = jnp.sqrt(jnp.maximum(d2, 0.0))
    idx_ref[0, 0, :] = jnp.argmin(d, axis=-1).astype(jnp.int32)


def _step_body(res_ref, qprev_ref, cb_ref, b2_ref, idx_ref, resout_ref,
               loss_ref):
    res = res_ref[...] - qprev_ref[...]
    resout_ref[...] = res
    a2 = jnp.sum(res * res, axis=-1, keepdims=True)
    loss_ref[...] = jnp.reshape(jnp.sum(a2), (1, 1, 1))
    scores = jax.lax.dot_general(
        res, cb_ref[...], (((1,), (1,)), ((), ())),
        preferred_element_type=jnp.float32)
    d2 = (a2 - 2.0 * scores) + b2_ref[...]
    d = jnp.sqrt(jnp.maximum(d2, 0.0))
    idx_ref[0, 0, :] = jnp.argmin(d, axis=-1).astype(jnp.int32)


_ROW_SPEC = pl.BlockSpec((TM, D), lambda m: (m, 0))
_CB_SPEC = pl.BlockSpec((K, D), lambda m: (0, 0))
_B2_SPEC = pl.BlockSpec((1, K), lambda m: (0, 0))
_IDX_SPEC = pl.BlockSpec((1, 1, TM), lambda m: (m, 0, 0))
_LOSS_SPEC = pl.BlockSpec((1, 1, 1), lambda m: (m, 0, 0))


def _first_step(x2d, cb, b2):
    return pl.pallas_call(
        _first_step_body,
        grid=(GRID_M,),
        in_specs=[_ROW_SPEC, _CB_SPEC, _B2_SPEC],
        out_specs=_IDX_SPEC,
        out_shape=jax.ShapeDtypeStruct((GRID_M, 1, TM), jnp.int32),
        compiler_params=_TC_PARAMS,
    )(x2d, cb, b2)


def _next_step(res_prev, qprev, cb, b2):
    return pl.pallas_call(
        _step_body,
        grid=(GRID_M,),
        in_specs=[_ROW_SPEC, _ROW_SPEC, _CB_SPEC, _B2_SPEC],
        out_specs=(_IDX_SPEC, _ROW_SPEC, _LOSS_SPEC),
        out_shape=(
            jax.ShapeDtypeStruct((GRID_M, 1, TM), jnp.int32),
            jax.ShapeDtypeStruct((M, D), jnp.float32),
            jax.ShapeDtypeStruct((GRID_M, 1, 1), jnp.float32),
        ),
        compiler_params=_TC_PARAMS,
    )(res_prev, qprev, cb, b2)


def _final_body(x_ref, q0_ref, q1_ref, q2_ref, q3_ref, quant_ref, loss_ref):
    q0, q1, q2, q3 = q0_ref[...], q1_ref[...], q2_ref[...], q3_ref[...]
    quant_ref[...] = ((q0 + q1) + q2) + q3
    res = (((x_ref[...] - q0) - q1) - q2) - q3
    loss_ref[...] = jnp.reshape(jnp.sum(res * res), (1, 1, 1))


def _final_step(x2d, qsteps):
    return pl.pallas_call(
        _final_body,
        grid=(GRID_M,),
        in_specs=[_ROW_SPEC] * 5,
        out_specs=(_ROW_SPEC, _LOSS_SPEC),
        out_shape=(
            jax.ShapeDtypeStruct((M, D), jnp.float32),
            jax.ShapeDtypeStruct((GRID_M, 1, 1), jnp.float32),
        ),
        compiler_params=_TC_PARAMS,
    )(x2d, *qsteps)


def _gather_rows(cb, idx):
    idx2 = idx.reshape(1, M)
    mesh = plsc.VectorSubcoreMesh(core_axis_name="core",
                                  subcore_axis_name="subcore")

    @pl.kernel(out_type=jax.ShapeDtypeStruct((M, D), jnp.float32), mesh=mesh)
    def kern(cb_hbm, i_hbm, o_hbm):
        def body(i_vmem, o_vmem):
            pltpu.sync_copy(cb_hbm.at[i_vmem.at[0]], o_vmem)

        pltpu.emit_pipeline(
            body,
            grid=(M // GW,),
            in_specs=[pl.BlockSpec((1, GW), index_map=lambda i: (0, i))],
            out_specs=[pl.BlockSpec((GW, D), index_map=lambda i: (i, 0))],
            core_axis_name=("core", "subcore"),
            dimension_semantics=(pltpu.PARALLEL,),
        )(i_hbm, o_hbm)

    return kern(cb, idx2)


def kernel(x, codebooks):
    x2d = x.reshape(M, D)
    b2 = _codebook_sqnorms(codebooks)

    idx0 = _first_step(x2d, codebooks[0], b2[0])
    q0 = _gather_rows(codebooks[0], idx0)

    res_prev = x2d
    qprev = q0
    all_idx = [idx0]
    qsteps = [q0]
    loss_parts = []
    for q in range(1, NQ):
        idx_q, res_q, loss_q = _next_step(res_prev, qprev, codebooks[q],
                                          b2[q])
        qstep = _gather_rows(codebooks[q], idx_q)
        all_idx.append(idx_q)
        qsteps.append(qstep)
        loss_parts.append(loss_q)
        res_prev, qprev = res_q, qstep

    quant, loss_last = _final_step(x2d, qsteps)
    loss_parts.append(loss_last)

    loss_sum = sum(jnp.sum(p) for p in loss_parts)
    commitment_loss = (loss_sum / jnp.float32(M * D)) * jnp.float32(0.25)

    indices = jnp.stack([i.reshape(M) for i in all_idx], axis=-1)
    return (quant.reshape(B, S, D), indices.reshape(B, S, NQ),
            commitment_loss)


# speedup vs baseline: 2.0513x; 1.6100x over previous
import jax
import jax.numpy as jnp
from jax.experimental import pallas as pl
from jax.experimental.pallas import tpu as pltpu
from jax.experimental.pallas import tpu_sc as plsc

NQ = 4
K = 8192
D = 256
B, S = 32, 576
M = B * S
TM = 256
GRID_M = M // TM
GW = 128

_TC_PARAMS = pltpu.CompilerParams(dimension_semantics=("parallel",))


def _b2_body(cb_ref, b2_ref):
    for q in range(NQ):
        cb = cb_ref[q]
        b2_ref[q] = jnp.sum(cb * cb, axis=-1)[None, :]


def _codebook_sqnorms(codebooks):
    return pl.pallas_call(
        _b2_body,
        out_shape=jax.ShapeDtypeStruct((NQ, 1, K), jnp.float32),
    )(codebooks)


def _ref_argmin(res, cb, b2):
    a2 = jnp.sum(res * res, axis=-1, keepdims=True)
    s2 = jax.lax.dot_general(
        jnp.float32(-2.0) * res, cb, (((1,), (1,)), ((), ())),
        preferred_element_type=jnp.float32)
    d2 = (a2 + s2) + b2
    m = jnp.min(d2, axis=-1, keepdims=True)
    iota = jax.lax.broadcasted_iota(jnp.int32, d2.shape, 1)
    cand = jnp.where(d2 <= m, iota, jnp.int32(K))
    return jnp.min(cand, axis=-1).astype(jnp.int32)


def _first_step_body(x_ref, cb_ref, b2_ref, idx_ref):
    res = x_ref[...]
    idx_ref[0, 0, :] = _ref_argmin(res, cb_ref[...], b2_ref[...])


def _step_body(res_ref, qprev_ref, cb_ref, b2_ref, idx_ref, resout_ref,
               loss_ref):
    res = res_ref[...] - qprev_ref[...]
    resout_ref[...] = res
    loss_ref[...] = jnp.reshape(jnp.sum(res * res), (1, 1, 1))
    idx_ref[0, 0, :] = _ref_argmin(res, cb_ref[...], b2_ref[...])


_ROW_SPEC = pl.BlockSpec((TM, D), lambda m: (m, 0))
_CB_SPEC = pl.BlockSpec((K, D), lambda m: (0, 0))
_B2_SPEC = pl.BlockSpec((1, K), lambda m: (0, 0))
_IDX_SPEC = pl.BlockSpec((1, 1, TM), lambda m: (m, 0, 0))
_LOSS_SPEC = pl.BlockSpec((1, 1, 1), lambda m: (m, 0, 0))


def _first_step(x2d, cb, b2):
    return pl.pallas_call(
        _first_step_body,
        grid=(GRID_M,),
        in_specs=[_ROW_SPEC, _CB_SPEC, _B2_SPEC],
        out_specs=_IDX_SPEC,
        out_shape=jax.ShapeDtypeStruct((GRID_M, 1, TM), jnp.int32),
        compiler_params=_TC_PARAMS,
    )(x2d, cb, b2)


def _next_step(res_prev, qprev, cb, b2):
    return pl.pallas_call(
        _step_body,
        grid=(GRID_M,),
        in_specs=[_ROW_SPEC, _ROW_SPEC, _CB_SPEC, _B2_SPEC],
        out_specs=(_IDX_SPEC, _ROW_SPEC, _LOSS_SPEC),
        out_shape=(
            jax.ShapeDtypeStruct((GRID_M, 1, TM), jnp.int32),
            jax.ShapeDtypeStruct((M, D), jnp.float32),
            jax.ShapeDtypeStruct((GRID_M, 1, 1), jnp.float32),
        ),
        compiler_params=_TC_PARAMS,
    )(res_prev, qprev, cb, b2)


def _final_body(x_ref, q0_ref, q1_ref, q2_ref, q3_ref, quant_ref, loss_ref):
    q0, q1, q2, q3 = q0_ref[...], q1_ref[...], q2_ref[...], q3_ref[...]
    quant_ref[...] = ((q0 + q1) + q2) + q3
    res = (((x_ref[...] - q0) - q1) - q2) - q3
    loss_ref[...] = jnp.reshape(jnp.sum(res * res), (1, 1, 1))


def _final_step(x2d, qsteps):
    return pl.pallas_call(
        _final_body,
        grid=(GRID_M,),
        in_specs=[_ROW_SPEC] * 5,
        out_specs=(_ROW_SPEC, _LOSS_SPEC),
        out_shape=(
            jax.ShapeDtypeStruct((M, D), jnp.float32),
            jax.ShapeDtypeStruct((GRID_M, 1, 1), jnp.float32),
        ),
        compiler_params=_TC_PARAMS,
    )(x2d, *qsteps)


def _gather_rows(cb, idx):
    idx2 = idx.reshape(1, M)
    mesh = plsc.VectorSubcoreMesh(core_axis_name="core",
                                  subcore_axis_name="subcore")

    @pl.kernel(out_type=jax.ShapeDtypeStruct((M, D), jnp.float32), mesh=mesh)
    def kern(cb_hbm, i_hbm, o_hbm):
        def body(i_vmem, o_vmem):
            pltpu.sync_copy(cb_hbm.at[i_vmem.at[0]], o_vmem)

        pltpu.emit_pipeline(
            body,
            grid=(M // GW,),
            in_specs=[pl.BlockSpec((1, GW), index_map=lambda i: (0, i))],
            out_specs=[pl.BlockSpec((GW, D), index_map=lambda i: (i, 0))],
            core_axis_name=("core", "subcore"),
            dimension_semantics=(pltpu.PARALLEL,),
        )(i_hbm, o_hbm)

    return kern(cb, idx2)


def kernel(x, codebooks):
    x2d = x.reshape(M, D)
    b2 = _codebook_sqnorms(codebooks)

    idx0 = _first_step(x2d, codebooks[0], b2[0])
    q0 = _gather_rows(codebooks[0], idx0)

    res_prev = x2d
    qprev = q0
    all_idx = [idx0]
    qsteps = [q0]
    loss_parts = []
    for q in range(1, NQ):
        idx_q, res_q, loss_q = _next_step(res_prev, qprev, codebooks[q],
                                          b2[q])
        qstep = _gather_rows(codebooks[q], idx_q)
        all_idx.append(idx_q)
        qsteps.append(qstep)
        loss_parts.append(loss_q)
        res_prev, qprev = res_q, qstep

    quant, loss_last = _final_step(x2d, qsteps)
    loss_parts.append(loss_last)

    loss_sum = sum(jnp.sum(p) for p in loss_parts)
    commitment_loss = (loss_sum / jnp.float32(M * D)) * jnp.float32(0.25)

    indices = jnp.stack([i.reshape(M) for i in all_idx], axis=-1)
    return (quant.reshape(B, S, D), indices.reshape(B, S, NQ),
            commitment_loss)


# speedup vs baseline: 2.5876x; 1.2614x over previous
import jax
import jax.numpy as jnp
from jax.experimental import pallas as pl
from jax.experimental.pallas import tpu as pltpu
from jax.experimental.pallas import tpu_sc as plsc

NQ = 4
K = 8192
D = 256
B, S = 32, 576
M = B * S
TM = 256
GRID_M = M // TM
GW = 128

_TC_PARAMS = pltpu.CompilerParams(dimension_semantics=("parallel",))


def _b2_body(cb_ref, b2_ref):
    for q in range(NQ):
        cb = cb_ref[q]
        b2_ref[q] = jnp.sum(cb * cb, axis=-1)[None, :]


def _codebook_sqnorms(codebooks):
    return pl.pallas_call(
        _b2_body,
        out_shape=jax.ShapeDtypeStruct((NQ, 1, K), jnp.float32),
    )(codebooks)


def _ref_argmin(res, cb, b2):
    a2 = jnp.sum(res * res, axis=-1, keepdims=True)
    s2 = jax.lax.dot_general(
        jnp.float32(-2.0) * res, cb, (((1,), (1,)), ((), ())),
        preferred_element_type=jnp.float32)
    d2 = (a2 + s2) + b2
    return jnp.argmin(d2, axis=-1).astype(jnp.int32)


def _first_step_body(x_ref, cb_ref, b2_ref, idx_ref):
    res = x_ref[...]
    idx_ref[0, 0, :] = _ref_argmin(res, cb_ref[...], b2_ref[...])


def _step_body(res_ref, qprev_ref, cb_ref, b2_ref, idx_ref, resout_ref,
               loss_ref):
    res = res_ref[...] - qprev_ref[...]
    resout_ref[...] = res
    loss_ref[...] = jnp.reshape(jnp.sum(res * res), (1, 1, 1))
    idx_ref[0, 0, :] = _ref_argmin(res, cb_ref[...], b2_ref[...])


_ROW_SPEC = pl.BlockSpec((TM, D), lambda m: (m, 0))
_CB_SPEC = pl.BlockSpec((K, D), lambda m: (0, 0))
_B2_SPEC = pl.BlockSpec((1, K), lambda m: (0, 0))
_IDX_SPEC = pl.BlockSpec((1, 1, TM), lambda m: (m, 0, 0))
_LOSS_SPEC = pl.BlockSpec((1, 1, 1), lambda m: (m, 0, 0))


def _first_step(x2d, cb, b2):
    return pl.pallas_call(
        _first_step_body,
        grid=(GRID_M,),
        in_specs=[_ROW_SPEC, _CB_SPEC, _B2_SPEC],
        out_specs=_IDX_SPEC,
        out_shape=jax.ShapeDtypeStruct((GRID_M, 1, TM), jnp.int32),
        compiler_params=_TC_PARAMS,
    )(x2d, cb, b2)


def _next_step(res_prev, qprev, cb, b2):
    return pl.pallas_call(
        _step_body,
        grid=(GRID_M,),
        in_specs=[_ROW_SPEC, _ROW_SPEC, _CB_SPEC, _B2_SPEC],
        out_specs=(_IDX_SPEC, _ROW_SPEC, _LOSS_SPEC),
        out_shape=(
            jax.ShapeDtypeStruct((GRID_M, 1, TM), jnp.int32),
            jax.ShapeDtypeStruct((M, D), jnp.float32),
            jax.ShapeDtypeStruct((GRID_M, 1, 1), jnp.float32),
        ),
        compiler_params=_TC_PARAMS,
    )(res_prev, qprev, cb, b2)


def _final_body(x_ref, q0_ref, q1_ref, q2_ref, q3_ref, quant_ref, loss_ref):
    q0, q1, q2, q3 = q0_ref[...], q1_ref[...], q2_ref[...], q3_ref[...]
    quant_ref[...] = ((q0 + q1) + q2) + q3
    res = (((x_ref[...] - q0) - q1) - q2) - q3
    loss_ref[...] = jnp.reshape(jnp.sum(res * res), (1, 1, 1))


def _final_step(x2d, qsteps):
    return pl.pallas_call(
        _final_body,
        grid=(GRID_M,),
        in_specs=[_ROW_SPEC] * 5,
        out_specs=(_ROW_SPEC, _LOSS_SPEC),
        out_shape=(
            jax.ShapeDtypeStruct((M, D), jnp.float32),
            jax.ShapeDtypeStruct((GRID_M, 1, 1), jnp.float32),
        ),
        compiler_params=_TC_PARAMS,
    )(x2d, *qsteps)


def _gather_rows(cb, idx):
    idx2 = idx.reshape(1, M)
    mesh = plsc.VectorSubcoreMesh(core_axis_name="core",
                                  subcore_axis_name="subcore")

    @pl.kernel(out_type=jax.ShapeDtypeStruct((M, D), jnp.float32), mesh=mesh)
    def kern(cb_hbm, i_hbm, o_hbm):
        def body(i_vmem, o_vmem):
            pltpu.sync_copy(cb_hbm.at[i_vmem.at[0]], o_vmem)

        pltpu.emit_pipeline(
            body,
            grid=(M // GW,),
            in_specs=[pl.BlockSpec((1, GW), index_map=lambda i: (0, i))],
            out_specs=[pl.BlockSpec((GW, D), index_map=lambda i: (i, 0))],
            core_axis_name=("core", "subcore"),
            dimension_semantics=(pltpu.PARALLEL,),
        )(i_hbm, o_hbm)

    return kern(cb, idx2)


def kernel(x, codebooks):
    x2d = x.reshape(M, D)
    b2 = _codebook_sqnorms(codebooks)

    idx0 = _first_step(x2d, codebooks[0], b2[0])
    q0 = _gather_rows(codebooks[0], idx0)

    res_prev = x2d
    qprev = q0
    all_idx = [idx0]
    qsteps = [q0]
    loss_parts = []
    for q in range(1, NQ):
        idx_q, res_q, loss_q = _next_step(res_prev, qprev, codebooks[q],
                                          b2[q])
        qstep = _gather_rows(codebooks[q], idx_q)
        all_idx.append(idx_q)
        qsteps.append(qstep)
        loss_parts.append(loss_q)
        res_prev, qprev = res_q, qstep

    quant, loss_last = _final_step(x2d, qsteps)
    loss_parts.append(loss_last)

    loss_sum = sum(jnp.sum(p) for p in loss_parts)
    commitment_loss = (loss_sum / jnp.float32(M * D)) * jnp.float32(0.25)

    indices = jnp.stack([i.reshape(M) for i in all_idx], axis=-1)
    return (quant.reshape(B, S, D), indices.reshape(B, S, NQ),
            commitment_loss)


# speedup vs baseline: 2.5986x; 1.0043x over previous
import jax
import jax.numpy as jnp
from jax.experimental import pallas as pl
from jax.experimental.pallas import tpu as pltpu
from jax.experimental.pallas import tpu_sc as plsc

NQ = 4
K = 8192
D = 256
B, S = 32, 576
M = B * S
TM = 256
MH = M // 2
GRID_H = MH // TM
GW = 128

_TC_PARAMS = pltpu.CompilerParams(dimension_semantics=("parallel",))


def _b2_body(cb_ref, b2_ref):
    for q in range(NQ):
        cb = cb_ref[q]
        b2_ref[q] = jnp.sum(cb * cb, axis=-1)[None, :]


def _codebook_sqnorms(codebooks):
    return pl.pallas_call(
        _b2_body,
        out_shape=jax.ShapeDtypeStruct((NQ, 1, K), jnp.float32),
    )(codebooks)


def _ref_argmin(res, cb, b2):
    a2 = jnp.sum(res * res, axis=-1, keepdims=True)
    s2 = jax.lax.dot_general(
        jnp.float32(-2.0) * res, cb, (((1,), (1,)), ((), ())),
        preferred_element_type=jnp.float32)
    d2 = (a2 + s2) + b2
    return jnp.argmin(d2, axis=-1).astype(jnp.int32)


def _first_step_body(x_ref, cb_ref, b2_ref, idx_ref):
    res = x_ref[...]
    idx_ref[0, 0, :] = _ref_argmin(res, cb_ref[...], b2_ref[...])


def _step_body(res_ref, qprev_ref, cb_ref, b2_ref, idx_ref, resout_ref,
               loss_ref):
    res = res_ref[...] - qprev_ref[...]
    resout_ref[...] = res
    loss_ref[...] = jnp.reshape(jnp.sum(res * res), (1, 1, 1))
    idx_ref[0, 0, :] = _ref_argmin(res, cb_ref[...], b2_ref[...])


_ROW_SPEC = pl.BlockSpec((TM, D), lambda m: (m, 0))
_CB_SPEC = pl.BlockSpec((K, D), lambda m: (0, 0))
_B2_SPEC = pl.BlockSpec((1, K), lambda m: (0, 0))
_IDX_SPEC = pl.BlockSpec((1, 1, TM), lambda m: (m, 0, 0))
_LOSS_SPEC = pl.BlockSpec((1, 1, 1), lambda m: (m, 0, 0))


def _first_step(xh, cb, b2):
    return pl.pallas_call(
        _first_step_body,
        grid=(GRID_H,),
        in_specs=[_ROW_SPEC, _CB_SPEC, _B2_SPEC],
        out_specs=_IDX_SPEC,
        out_shape=jax.ShapeDtypeStruct((GRID_H, 1, TM), jnp.int32),
        compiler_params=_TC_PARAMS,
    )(xh, cb, b2)


def _next_step(res_prev, qprev, cb, b2):
    return pl.pallas_call(
        _step_body,
        grid=(GRID_H,),
        in_specs=[_ROW_SPEC, _ROW_SPEC, _CB_SPEC, _B2_SPEC],
        out_specs=(_IDX_SPEC, _ROW_SPEC, _LOSS_SPEC),
        out_shape=(
            jax.ShapeDtypeStruct((GRID_H, 1, TM), jnp.int32),
            jax.ShapeDtypeStruct((MH, D), jnp.float32),
            jax.ShapeDtypeStruct((GRID_H, 1, 1), jnp.float32),
        ),
        compiler_params=_TC_PARAMS,
    )(res_prev, qprev, cb, b2)


def _final_body(x_ref, q0_ref, q1_ref, q2_ref, q3_ref, quant_ref, loss_ref):
    q0, q1, q2, q3 = q0_ref[...], q1_ref[...], q2_ref[...], q3_ref[...]
    quant_ref[...] = ((q0 + q1) + q2) + q3
    res = (((x_ref[...] - q0) - q1) - q2) - q3
    loss_ref[...] = jnp.reshape(jnp.sum(res * res), (1, 1, 1))


def _final_step(xh, qsteps):
    return pl.pallas_call(
        _final_body,
        grid=(GRID_H,),
        in_specs=[_ROW_SPEC] * 5,
        out_specs=(_ROW_SPEC, _LOSS_SPEC),
        out_shape=(
            jax.ShapeDtypeStruct((MH, D), jnp.float32),
            jax.ShapeDtypeStruct((GRID_H, 1, 1), jnp.float32),
        ),
        compiler_params=_TC_PARAMS,
    )(xh, *qsteps)


def _gather_rows(cb, idx):
    idx2 = idx.reshape(1, MH)
    mesh = plsc.VectorSubcoreMesh(core_axis_name="core",
                                  subcore_axis_name="subcore")

    @pl.kernel(out_type=jax.ShapeDtypeStruct((MH, D), jnp.float32), mesh=mesh)
    def kern(cb_hbm, i_hbm, o_hbm):
        def body(i_vmem, o_vmem):
            pltpu.sync_copy(cb_hbm.at[i_vmem.at[0]], o_vmem)

        pltpu.emit_pipeline(
            body,
            grid=(MH // GW,),
            in_specs=[pl.BlockSpec((1, GW), index_map=lambda i: (0, i))],
            out_specs=[pl.BlockSpec((GW, D), index_map=lambda i: (i, 0))],
            core_axis_name=("core", "subcore"),
            dimension_semantics=(pltpu.PARALLEL,),
        )(i_hbm, o_hbm)

    return kern(cb, idx2)


def kernel(x, codebooks):
    x2d = x.reshape(M, D)
    b2 = _codebook_sqnorms(codebooks)
    halves = (x2d[:MH], x2d[MH:])

    res = list(halves)
    qprev = [None, None]
    all_idx = [[], []]
    qsteps = [[], []]
    loss_parts = []
    for q in range(NQ):
        cb, b2q = codebooks[q], b2[q]
        idx_h = [None, None]
        for h in range(2):
            if q == 0:
                idx_h[h] = _first_step(res[h], cb, b2q)
            else:
                idx_h[h], res_h, loss_h = _next_step(res[h], qprev[h], cb,
                                                     b2q)
                res[h] = res_h
                loss_parts.append(loss_h)
            all_idx[h].append(idx_h[h])
        for h in range(2):
            g = _gather_rows(cb, idx_h[h])
            qprev[h] = g
            qsteps[h].append(g)

    quant_h = []
    for h in range(2):
        quant, loss_last = _final_step(halves[h], qsteps[h])
        quant_h.append(quant)
        loss_parts.append(loss_last)

    loss_sum = sum(jnp.sum(p) for p in loss_parts)
    commitment_loss = (loss_sum / jnp.float32(M * D)) * jnp.float32(0.25)

    quant = jnp.concatenate(quant_h, axis=0)
    indices = jnp.stack(
        [jnp.concatenate([a.reshape(MH), b.reshape(MH)])
         for a, b in zip(all_idx[0], all_idx[1])], axis=-1)
    return (quant.reshape(B, S, D), indices.reshape(B, S, NQ),
            commitment_loss)
